# baseline (speedup 1.0000x reference)
import functools

import jax
import jax.numpy as jnp
from jax.experimental import pallas as pl
from jax.experimental.pallas import tpu as pltpu

_SLOPE = 0.02
_EPS = 1e-5


def _conv_stats_kernel(x_ref, w_ref, y_ref, s1_ref, s2_ref, *, H, W):
    NB, C, HW = x_ref.shape

    x = x_ref[...]
    a = jnp.where(x >= 0, x, x * _SLOPE)

    p = jax.lax.broadcasted_iota(jnp.int32, (1, 1, HW), 2)
    w_of_p = p % W

    segs = []
    for kh in range(3):
        for kw in range(3):
            d = (kh - 1) * W + (kw - 1)
            if d > 0:
                s = jnp.concatenate(
                    [a[:, :, d:], jnp.zeros((NB, C, d), a.dtype)], axis=2)
            elif d < 0:
                s = jnp.concatenate(
                    [jnp.zeros((NB, C, -d), a.dtype), a[:, :, :d]], axis=2)
            else:
                s = a
            m = jnp.ones((1, 1, HW), dtype=jnp.bool_)
            if kh == 0:
                m = m & (p >= W)
            elif kh == 2:
                m = m & (p < HW - W)
            if kw == 0:
                m = m & (w_of_p >= 1)
            elif kw == 2:
                m = m & (w_of_p < W - 1)
            segs.append(jnp.where(m, s, jnp.zeros_like(s)))
    a9 = jnp.concatenate(segs, axis=1)

    w9 = w_ref[...]
    s1 = jnp.zeros((C, HW), jnp.float32)
    s2 = jnp.zeros((C, HW), jnp.float32)
    for i in range(NB):
        y = jax.lax.dot_general(
            w9, a9[i], (((0,), (0,)), ((), ())),
            preferred_element_type=jnp.float32)
        y_ref[i] = y.astype(y_ref.dtype)
        s1 = s1 + y
        s2 = s2 + y * y
    s1_ref[...] = jnp.sum(s1, axis=1, keepdims=True).reshape(1, C, 1)
    s2_ref[...] = jnp.sum(s2, axis=1, keepdims=True).reshape(1, C, 1)


def _bn_apply_kernel(y_ref, scale_ref, shift_ref, out_ref):
    out_ref[...] = (y_ref[...].astype(jnp.float32) * scale_ref[...]
                    + shift_ref[...]).astype(out_ref.dtype)


@jax.jit
def _forward(x_nchw, w_deconv, gamma, beta):
    N, C, H, W = x_nchw.shape
    HW = H * W
    NB = 16
    while N % NB:
        NB //= 2
    G = N // NB

    x3 = x_nchw.reshape(N, C, HW).astype(jnp.bfloat16)
    wc = jnp.transpose(jnp.flip(w_deconv, axis=(2, 3)), (2, 3, 0, 1))
    w9 = wc.reshape(9 * C, C).astype(jnp.bfloat16)

    cparams = pltpu.CompilerParams(
        dimension_semantics=("parallel",),
        vmem_limit_bytes=64 * 1024 * 1024,
    )

    conv_kernel = functools.partial(_conv_stats_kernel, H=H, W=W)
    y, s1, s2 = pl.pallas_call(
        conv_kernel,
        grid=(G,),
        in_specs=[
            pl.BlockSpec((NB, C, HW), lambda g: (g, 0, 0)),
            pl.BlockSpec((9 * C, C), lambda g: (0, 0)),
        ],
        out_specs=(
            pl.BlockSpec((NB, C, HW), lambda g: (g, 0, 0)),
            pl.BlockSpec((1, C, 1), lambda g: (g, 0, 0)),
            pl.BlockSpec((1, C, 1), lambda g: (g, 0, 0)),
        ),
        out_shape=(
            jax.ShapeDtypeStruct((N, C, HW), jnp.bfloat16),
            jax.ShapeDtypeStruct((G, C, 1), jnp.float32),
            jax.ShapeDtypeStruct((G, C, 1), jnp.float32),
        ),
        compiler_params=cparams,
    )(x3, w9)

    m_total = float(N * H * W)
    sum_c = jnp.sum(s1, axis=(0, 2))
    sq_c = jnp.sum(s2, axis=(0, 2))
    mean = sum_c / m_total
    var = jnp.maximum(sq_c / m_total - mean * mean, 0.0)
    inv = jax.lax.rsqrt(var + _EPS)
    scale_c = gamma.astype(jnp.float32) * inv
    shift_c = beta.astype(jnp.float32) - mean * scale_c
    scale_b = jnp.broadcast_to(scale_c[:, None], (C, HW))
    shift_b = jnp.broadcast_to(shift_c[:, None], (C, HW))

    out3 = pl.pallas_call(
        _bn_apply_kernel,
        grid=(G,),
        in_specs=[
            pl.BlockSpec((NB, C, HW), lambda g: (g, 0, 0)),
            pl.BlockSpec((C, HW), lambda g: (0, 0)),
            pl.BlockSpec((C, HW), lambda g: (0, 0)),
        ],
        out_specs=pl.BlockSpec((NB, C, HW), lambda g: (g, 0, 0)),
        out_shape=jax.ShapeDtypeStruct((N, C, HW), jnp.bfloat16),
        compiler_params=cparams,
    )(y, scale_b, shift_b)

    return out3.reshape(N, C, H, W).astype(jnp.float32)


def kernel(x_nchw, w_deconv, gamma, beta):
    return _forward(x_nchw, w_deconv, gamma, beta)


# speedup vs baseline: 1.1989x; 1.1989x over previous
import functools

import jax
import jax.numpy as jnp
from jax.experimental import pallas as pl
from jax.experimental.pallas import tpu as pltpu

_SLOPE = 0.02
_EPS = 1e-5


def _conv_stats_kernel(x_ref, w_ref, y_ref, s1_ref, s2_ref, *, H, W):
    NB, C, HW = x_ref.shape
    CH = min(16, NB)

    p = jax.lax.broadcasted_iota(jnp.int32, (1, 1, HW), 2)
    w_of_p = p % W
    w9 = w_ref[...]

    s1 = jnp.zeros((C, HW), jnp.float32)
    s2 = jnp.zeros((C, HW), jnp.float32)
    for i0 in range(0, NB, CH):
        x = x_ref[i0:i0 + CH].astype(jnp.bfloat16)
        a = jnp.where(x >= 0, x, x * _SLOPE)

        segs = []
        for kh in range(3):
            for kw in range(3):
                d = (kh - 1) * W + (kw - 1)
                if d > 0:
                    s = jnp.concatenate(
                        [a[:, :, d:], jnp.zeros((CH, C, d), a.dtype)], axis=2)
                elif d < 0:
                    s = jnp.concatenate(
                        [jnp.zeros((CH, C, -d), a.dtype), a[:, :, :d]], axis=2)
                else:
                    s = a
                m = jnp.ones((1, 1, HW), dtype=jnp.bool_)
                if kh == 0:
                    m = m & (p >= W)
                elif kh == 2:
                    m = m & (p < HW - W)
                if kw == 0:
                    m = m & (w_of_p >= 1)
                elif kw == 2:
                    m = m & (w_of_p < W - 1)
                segs.append(jnp.where(m, s, jnp.zeros_like(s)))
        a9 = jnp.concatenate(segs, axis=1)

        for i in range(CH):
            y = jax.lax.dot_general(
                w9, a9[i], (((0,), (0,)), ((), ())),
                preferred_element_type=jnp.float32)
            y_ref[i0 + i] = y.astype(y_ref.dtype)
            s1 = s1 + y
            s2 = s2 + y * y
    s1_ref[...] = jnp.sum(s1, axis=1, keepdims=True).reshape(1, C, 1)
    s2_ref[...] = jnp.sum(s2, axis=1, keepdims=True).reshape(1, C, 1)


def _bn_apply_kernel(y_ref, scale_ref, shift_ref, out_ref):
    out_ref[...] = (y_ref[...].astype(jnp.float32) * scale_ref[...]
                    + shift_ref[...]).astype(out_ref.dtype)


@jax.jit
def _forward(x_nchw, w_deconv, gamma, beta):
    N, C, H, W = x_nchw.shape
    HW = H * W
    NB = 64
    while N % NB:
        NB //= 2
    G = N // NB

    x3 = x_nchw.reshape(N, C, HW)
    wc = jnp.transpose(jnp.flip(w_deconv, axis=(2, 3)), (2, 3, 0, 1))
    w9 = wc.reshape(9 * C, C).astype(jnp.bfloat16)

    cparams = pltpu.CompilerParams(
        dimension_semantics=("parallel",),
        vmem_limit_bytes=64 * 1024 * 1024,
    )

    conv_kernel = functools.partial(_conv_stats_kernel, H=H, W=W)
    y, s1, s2 = pl.pallas_call(
        conv_kernel,
        grid=(G,),
        in_specs=[
            pl.BlockSpec((NB, C, HW), lambda g: (g, 0, 0)),
            pl.BlockSpec((9 * C, C), lambda g: (0, 0)),
        ],
        out_specs=(
            pl.BlockSpec((NB, C, HW), lambda g: (g, 0, 0)),
            pl.BlockSpec((1, C, 1), lambda g: (g, 0, 0)),
            pl.BlockSpec((1, C, 1), lambda g: (g, 0, 0)),
        ),
        out_shape=(
            jax.ShapeDtypeStruct((N, C, HW), jnp.bfloat16),
            jax.ShapeDtypeStruct((G, C, 1), jnp.float32),
            jax.ShapeDtypeStruct((G, C, 1), jnp.float32),
        ),
        compiler_params=cparams,
    )(x3, w9)

    m_total = float(N * H * W)
    sum_c = jnp.sum(s1, axis=(0, 2))
    sq_c = jnp.sum(s2, axis=(0, 2))
    mean = sum_c / m_total
    var = jnp.maximum(sq_c / m_total - mean * mean, 0.0)
    inv = jax.lax.rsqrt(var + _EPS)
    scale_c = gamma.astype(jnp.float32) * inv
    shift_c = beta.astype(jnp.float32) - mean * scale_c
    scale_b = jnp.broadcast_to(scale_c[:, None], (C, HW))
    shift_b = jnp.broadcast_to(shift_c[:, None], (C, HW))

    out3 = pl.pallas_call(
        _bn_apply_kernel,
        grid=(G,),
        in_specs=[
            pl.BlockSpec((NB, C, HW), lambda g: (g, 0, 0)),
            pl.BlockSpec((C, HW), lambda g: (0, 0)),
            pl.BlockSpec((C, HW), lambda g: (0, 0)),
        ],
        out_specs=pl.BlockSpec((NB, C, HW), lambda g: (g, 0, 0)),
        out_shape=jax.ShapeDtypeStruct((N, C, HW), jnp.bfloat16),
        compiler_params=cparams,
    )(y, scale_b, shift_b)

    return out3.reshape(N, C, H, W).astype(jnp.float32)


def kernel(x_nchw, w_deconv, gamma, beta):
    return _forward(x_nchw, w_deconv, gamma, beta)


# speedup vs baseline: 1.2737x; 1.0624x over previous
import functools

import jax
import jax.numpy as jnp
from jax.experimental import pallas as pl
from jax.experimental.pallas import tpu as pltpu

_SLOPE = 0.02
_EPS = 1e-5


def _reformat_kernel(x_ref, a_ref):
    NBL = x_ref.shape[1]
    C, HW = a_ref.shape[1], a_ref.shape[2]
    x = x_ref[...].astype(jnp.bfloat16)
    a = jnp.where(x >= 0, x, x * _SLOPE)
    a_ref[...] = jnp.transpose(a).reshape(NBL, C, HW)


def _conv_stats_kernel(x_ref, w_ref, y_ref, s1_ref, s2_ref, *, H, W):
    NB, C, HW = x_ref.shape
    CH = min(16, NB)

    p = jax.lax.broadcasted_iota(jnp.int32, (1, 1, HW), 2)
    w_of_p = p % W
    w9 = w_ref[...]

    s1 = jnp.zeros((C, HW), jnp.float32)
    s2 = jnp.zeros((C, HW), jnp.float32)
    for i0 in range(0, NB, CH):
        a = x_ref[i0:i0 + CH]

        segs = []
        for kh in range(3):
            for kw in range(3):
                d = (kh - 1) * W + (kw - 1)
                if d > 0:
                    s = jnp.concatenate(
                        [a[:, :, d:], jnp.zeros((CH, C, d), a.dtype)], axis=2)
                elif d < 0:
                    s = jnp.concatenate(
                        [jnp.zeros((CH, C, -d), a.dtype), a[:, :, :d]], axis=2)
                else:
                    s = a
                m = jnp.ones((1, 1, HW), dtype=jnp.bool_)
                if kh == 0:
                    m = m & (p >= W)
                elif kh == 2:
                    m = m & (p < HW - W)
                if kw == 0:
                    m = m & (w_of_p >= 1)
                elif kw == 2:
                    m = m & (w_of_p < W - 1)
                segs.append(jnp.where(m, s, jnp.zeros_like(s)))
        a9 = jnp.concatenate(segs, axis=1)

        for i in range(CH):
            y = jax.lax.dot_general(
                w9, a9[i], (((0,), (0,)), ((), ())),
                preferred_element_type=jnp.float32)
            y_ref[i0 + i] = y.astype(y_ref.dtype)
            s1 = s1 + y
            s2 = s2 + y * y
    s1_ref[...] = jnp.sum(s1, axis=1, keepdims=True).reshape(1, C, 1)
    s2_ref[...] = jnp.sum(s2, axis=1, keepdims=True).reshape(1, C, 1)


def _bn_apply_kernel(y_ref, scale_ref, shift_ref, out_ref):
    out_ref[...] = (y_ref[...].astype(jnp.float32) * scale_ref[...]
                    + shift_ref[...]).astype(out_ref.dtype)


@jax.jit
def _forward(x_nchw, w_deconv, gamma, beta):
    N, C, H, W = x_nchw.shape
    HW = H * W
    NB = 64
    while N % NB:
        NB //= 2
    G = N // NB
    NBL = 128
    while N % NBL:
        NBL //= 2
    GL = N // NBL

    cparams_fmt = pltpu.CompilerParams(
        dimension_semantics=("parallel",),
        vmem_limit_bytes=64 * 1024 * 1024,
    )
    xv = jnp.transpose(x_nchw, (1, 2, 3, 0)).reshape(C * HW, N)
    x3 = pl.pallas_call(
        _reformat_kernel,
        grid=(GL,),
        in_specs=[pl.BlockSpec((C * HW, NBL), lambda g: (0, g))],
        out_specs=pl.BlockSpec((NBL, C, HW), lambda g: (g, 0, 0)),
        out_shape=jax.ShapeDtypeStruct((N, C, HW), jnp.bfloat16),
        compiler_params=cparams_fmt,
    )(xv)
    wc = jnp.transpose(jnp.flip(w_deconv, axis=(2, 3)), (2, 3, 0, 1))
    w9 = wc.reshape(9 * C, C).astype(jnp.bfloat16)

    cparams = pltpu.CompilerParams(
        dimension_semantics=("parallel",),
        vmem_limit_bytes=64 * 1024 * 1024,
    )

    conv_kernel = functools.partial(_conv_stats_kernel, H=H, W=W)
    y, s1, s2 = pl.pallas_call(
        conv_kernel,
        grid=(G,),
        in_specs=[
            pl.BlockSpec((NB, C, HW), lambda g: (g, 0, 0)),
            pl.BlockSpec((9 * C, C), lambda g: (0, 0)),
        ],
        out_specs=(
            pl.BlockSpec((NB, C, HW), lambda g: (g, 0, 0)),
            pl.BlockSpec((1, C, 1), lambda g: (g, 0, 0)),
            pl.BlockSpec((1, C, 1), lambda g: (g, 0, 0)),
        ),
        out_shape=(
            jax.ShapeDtypeStruct((N, C, HW), jnp.bfloat16),
            jax.ShapeDtypeStruct((G, C, 1), jnp.float32),
            jax.ShapeDtypeStruct((G, C, 1), jnp.float32),
        ),
        compiler_params=cparams,
    )(x3, w9)

    m_total = float(N * H * W)
    sum_c = jnp.sum(s1, axis=(0, 2))
    sq_c = jnp.sum(s2, axis=(0, 2))
    mean = sum_c / m_total
    var = jnp.maximum(sq_c / m_total - mean * mean, 0.0)
    inv = jax.lax.rsqrt(var + _EPS)
    scale_c = gamma.astype(jnp.float32) * inv
    shift_c = beta.astype(jnp.float32) - mean * scale_c
    scale_b = jnp.broadcast_to(scale_c[:, None], (C, HW))
    shift_b = jnp.broadcast_to(shift_c[:, None], (C, HW))

    out3 = pl.pallas_call(
        _bn_apply_kernel,
        grid=(G,),
        in_specs=[
            pl.BlockSpec((NB, C, HW), lambda g: (g, 0, 0)),
            pl.BlockSpec((C, HW), lambda g: (0, 0)),
            pl.BlockSpec((C, HW), lambda g: (0, 0)),
        ],
        out_specs=pl.BlockSpec((NB, C, HW), lambda g: (g, 0, 0)),
        out_shape=jax.ShapeDtypeStruct((N, C, HW), jnp.float32),
        compiler_params=cparams,
    )(y, scale_b, shift_b)

    return out3.reshape(N, C, H, W)


def kernel(x_nchw, w_deconv, gamma, beta):
    return _forward(x_nchw, w_deconv, gamma, beta)


# speedup vs baseline: 1.5128x; 1.1877x over previous
import functools

import jax
import jax.numpy as jnp
from jax.experimental import pallas as pl
from jax.experimental.pallas import tpu as pltpu

_SLOPE = 0.02
_EPS = 1e-5


def _conv_stats_kernel(x_ref, w_ref, y_ref, s1_ref, s2_ref, *, H, W):
    NBL = x_ref.shape[1]
    _, C, HW = y_ref.shape
    CH = min(16, NBL)

    xv = x_ref[...].astype(jnp.bfloat16)
    av = jnp.where(xv >= 0, xv, xv * _SLOPE)
    a_all = jnp.transpose(av).reshape(NBL, C, HW)

    p = jax.lax.broadcasted_iota(jnp.int32, (1, 1, HW), 2)
    w_of_p = p % W
    w9 = w_ref[...]

    s1 = jnp.zeros((C, HW), jnp.float32)
    s2 = jnp.zeros((C, HW), jnp.float32)
    for i0 in range(0, NBL, CH):
        a = a_all[i0:i0 + CH]

        segs = []
        for kh in range(3):
            for kw in range(3):
                d = (kh - 1) * W + (kw - 1)
                if d > 0:
                    s = jnp.concatenate(
                        [a[:, :, d:], jnp.zeros((CH, C, d), a.dtype)], axis=2)
                elif d < 0:
                    s = jnp.concatenate(
                        [jnp.zeros((CH, C, -d), a.dtype), a[:, :, :d]], axis=2)
                else:
                    s = a
                m = jnp.ones((1, 1, HW), dtype=jnp.bool_)
                if kh == 0:
                    m = m & (p >= W)
                elif kh == 2:
                    m = m & (p < HW - W)
                if kw == 0:
                    m = m & (w_of_p >= 1)
                elif kw == 2:
                    m = m & (w_of_p < W - 1)
                segs.append(jnp.where(m, s, jnp.zeros_like(s)))
        a9 = jnp.concatenate(segs, axis=1)

        for i in range(CH):
            y = jax.lax.dot_general(
                w9, a9[i], (((0,), (0,)), ((), ())),
                preferred_element_type=jnp.float32)
            y_ref[i0 + i] = y.astype(y_ref.dtype)
            s1 = s1 + y
            s2 = s2 + y * y
    s1_ref[...] = jnp.sum(s1, axis=1, keepdims=True).reshape(1, C, 1)
    s2_ref[...] = jnp.sum(s2, axis=1, keepdims=True).reshape(1, C, 1)


def _bn_apply_kernel(y_ref, scale_ref, shift_ref, out_ref):
    NBL, C, HW = y_ref.shape
    yt = jnp.transpose(y_ref[...].reshape(NBL, C * HW))
    out_ref[...] = (yt.astype(jnp.float32) * scale_ref[...]
                    + shift_ref[...])


@jax.jit
def _forward(x_nchw, w_deconv, gamma, beta):
    N, C, H, W = x_nchw.shape
    HW = H * W
    CHW = C * HW
    NBL = 128
    while N % NBL:
        NBL //= 2
    GL = N // NBL

    xv = jnp.transpose(x_nchw, (1, 2, 3, 0)).reshape(CHW, N)
    wc = jnp.transpose(jnp.flip(w_deconv, axis=(2, 3)), (2, 3, 0, 1))
    w9 = wc.reshape(9 * C, C).astype(jnp.bfloat16)

    cparams = pltpu.CompilerParams(
        dimension_semantics=("parallel",),
        vmem_limit_bytes=64 * 1024 * 1024,
    )

    conv_kernel = functools.partial(_conv_stats_kernel, H=H, W=W)
    y, s1, s2 = pl.pallas_call(
        conv_kernel,
        grid=(GL,),
        in_specs=[
            pl.BlockSpec((CHW, NBL), lambda g: (0, g)),
            pl.BlockSpec((9 * C, C), lambda g: (0, 0)),
        ],
        out_specs=(
            pl.BlockSpec((NBL, C, HW), lambda g: (g, 0, 0)),
            pl.BlockSpec((1, C, 1), lambda g: (g, 0, 0)),
            pl.BlockSpec((1, C, 1), lambda g: (g, 0, 0)),
        ),
        out_shape=(
            jax.ShapeDtypeStruct((N, C, HW), jnp.bfloat16),
            jax.ShapeDtypeStruct((GL, C, 1), jnp.float32),
            jax.ShapeDtypeStruct((GL, C, 1), jnp.float32),
        ),
        compiler_params=cparams,
    )(xv, w9)

    m_total = float(N * H * W)
    sum_c = jnp.sum(s1, axis=(0, 2))
    sq_c = jnp.sum(s2, axis=(0, 2))
    mean = sum_c / m_total
    var = jnp.maximum(sq_c / m_total - mean * mean, 0.0)
    inv = jax.lax.rsqrt(var + _EPS)
    scale_c = gamma.astype(jnp.float32) * inv
    shift_c = beta.astype(jnp.float32) - mean * scale_c
    scale_r = jnp.repeat(scale_c, HW).reshape(CHW, 1)
    shift_r = jnp.repeat(shift_c, HW).reshape(CHW, 1)

    out_v = pl.pallas_call(
        _bn_apply_kernel,
        grid=(GL,),
        in_specs=[
            pl.BlockSpec((NBL, C, HW), lambda g: (g, 0, 0)),
            pl.BlockSpec((CHW, 1), lambda g: (0, 0)),
            pl.BlockSpec((CHW, 1), lambda g: (0, 0)),
        ],
        out_specs=pl.BlockSpec((CHW, NBL), lambda g: (0, g)),
        out_shape=jax.ShapeDtypeStruct((CHW, N), jnp.float32),
        compiler_params=cparams,
    )(y, scale_r, shift_r)

    return jnp.transpose(out_v.reshape(C, H, W, N), (3, 0, 1, 2))


def kernel(x_nchw, w_deconv, gamma, beta):
    return _forward(x_nchw, w_deconv, gamma, beta)
